# Initial kernel scaffold; baseline (speedup 1.0000x reference)
import jax
import jax.numpy as jnp
from jax.experimental import pallas as pl


def kernel(user_ids, item_ids, user_table, item_table):
    raise NotImplementedError("write your pallas kernel here")


# speedup vs baseline: 1.1019x; 1.1019x over previous
import functools

import jax
import jax.numpy as jnp
from jax import lax
from jax.experimental import pallas as pl
from jax.experimental.pallas import tpu as pltpu
from jax.experimental.pallas import tpu_sc as plsc

_B = 4096
_D = 32
_NC = 2
_NS = 16
_NW = _NC * _NS
_BPW = _B // _NW

_BLK = 512


@functools.lru_cache(maxsize=1)
def _make_sc_gather():
  mesh = plsc.VectorSubcoreMesh(core_axis_name="c", subcore_axis_name="s")

  @functools.partial(
      pl.kernel,
      mesh=mesh,
      out_type=(
          jax.ShapeDtypeStruct((_B, _D), jnp.float32),
          jax.ShapeDtypeStruct((_B, _D), jnp.float32),
      ),
      scratch_types=[
          pltpu.VMEM((_BPW,), jnp.int32),
          pltpu.VMEM((_BPW, _D), jnp.float32),
          pltpu.SemaphoreType.DMA,
      ],
      compiler_params=pltpu.CompilerParams(use_tc_tiling_on_sc=False),
  )
  def gather2(uid_hbm, iid_hbm, ut_hbm, it_hbm, u_out, p_out,
              idx_v, rows_v, sem):
    wid = lax.axis_index("s") * _NC + lax.axis_index("c")
    base = wid * _BPW
    pltpu.sync_copy(uid_hbm.at[pl.ds(base, _BPW)], idx_v)
    pltpu.async_copy(ut_hbm.at[idx_v], rows_v, sem).wait()
    pltpu.sync_copy(rows_v, u_out.at[pl.ds(base, _BPW)])
    pltpu.sync_copy(iid_hbm.at[pl.ds(base, _BPW)], idx_v)
    pltpu.async_copy(it_hbm.at[idx_v], rows_v, sem).wait()
    pltpu.sync_copy(rows_v, p_out.at[pl.ds(base, _BPW)])

  return gather2


def _loss_body(u_ref, p_ref, out_ref):
  i = pl.program_id(0)
  u = u_ref[...]
  p = p_ref[...]
  s = lax.dot_general(
      u, p, (((1,), (1,)), ((), ())),
      preferred_element_type=jnp.float32,
      precision=lax.Precision.HIGHEST,
  )
  m = jnp.max(s, axis=1, keepdims=True)
  lse = m + jnp.log(jnp.sum(jnp.exp(s - m), axis=1, keepdims=True))
  row = lax.broadcasted_iota(jnp.int32, (_BLK, _B), 0)
  col = lax.broadcasted_iota(jnp.int32, (_BLK, _B), 1)
  diag = jnp.sum(
      jnp.where(col == row + i * _BLK, s, 0.0), axis=1, keepdims=True)
  part = jnp.sum(lse - diag)

  @pl.when(i == 0)
  def _():
    out_ref[0, 0] = 0.0

  out_ref[0, 0] += part


_loss_call = pl.pallas_call(
    _loss_body,
    grid=(_B // _BLK,),
    in_specs=[
        pl.BlockSpec((_BLK, _D), lambda i: (i, 0)),
        pl.BlockSpec((_B, _D), lambda i: (0, 0)),
    ],
    out_specs=pl.BlockSpec(memory_space=pltpu.SMEM),
    out_shape=jax.ShapeDtypeStruct((1, 1), jnp.float32),
)


@jax.jit
def kernel(user_ids, item_ids, user_table, item_table):
  u, p = _make_sc_gather()(user_ids.astype(jnp.int32),
                           item_ids.astype(jnp.int32),
                           user_table, item_table)
  loss = _loss_call(u, p)
  return loss[0, 0]
